# baseline (speedup 1.0000x reference)
import functools

import jax
import jax.numpy as jnp
from jax import lax
from jax.experimental import pallas as pl
from jax.experimental.pallas import tpu as pltpu
from jax.experimental.pallas import tpu_sc as plsc

N = 10000
E = 320000
IN_FEAT = 128
OUT_FEAT = 128
NUM_RELS = 8
NUM_BASES = 4

NC = 2
NS = 16
NW = NC * NS
LANES = 16

EDGES_PER_WORKER = E // NW
CHUNK = 80
SBLOCK = 2000
CHUNKS_PER_SBLOCK = SBLOCK // CHUNK
NBLOCKS = EDGES_PER_WORKER // SBLOCK
ACC_ROWS = 10240
ROWS_PER_SUB = ACC_ROWS // NS


_BN = 1000


def _xw_body(x_ref, w2d_ref, wc_ref, out_ref, wbig_ref):
    @pl.when(pl.program_id(0) == 0)
    def _build_w():
        ki = lax.broadcasted_iota(jnp.int32, (NUM_RELS * IN_FEAT, NUM_BASES * IN_FEAT), 0)
        ji = lax.broadcasted_iota(jnp.int32, (NUM_RELS * IN_FEAT, NUM_BASES * IN_FEAT), 1)
        blk = (ki // NUM_RELS) == (ji // NUM_BASES)
        r_idx = lax.rem(ki, NUM_RELS)
        b_idx = lax.rem(ji, NUM_BASES)
        acc = jnp.zeros(ki.shape, jnp.float32)
        for r in range(NUM_RELS):
            for b in range(NUM_BASES):
                m = (r_idx == r) & (b_idx == b)
                acc = acc + jnp.where(m, wc_ref[r, b], 0.0)
        mmat = jnp.where(blk, acc, 0.0)
        wbig_ref[...] = jnp.dot(mmat, w2d_ref[...], preferred_element_type=jnp.float32)

    x = x_ref[...]
    for r in range(NUM_RELS):
        out_ref[:, OUT_FEAT * r:OUT_FEAT * (r + 1)] = jnp.dot(
            x, wbig_ref[IN_FEAT * r:IN_FEAT * (r + 1), :],
            preferred_element_type=jnp.float32)


def _xw_table(x, w2d, w_comp):
    return pl.pallas_call(
        _xw_body,
        grid=(N // _BN,),
        in_specs=[
            pl.BlockSpec((_BN, IN_FEAT), lambda i: (i, 0)),
            pl.BlockSpec((NUM_BASES * IN_FEAT, OUT_FEAT), lambda i: (0, 0)),
            pl.BlockSpec(memory_space=pltpu.SMEM),
        ],
        out_specs=pl.BlockSpec((_BN, NUM_RELS * OUT_FEAT), lambda i: (i, 0)),
        out_shape=jax.ShapeDtypeStruct((N, NUM_RELS * OUT_FEAT), jnp.float32),
        scratch_shapes=[pltpu.VMEM((NUM_RELS * IN_FEAT, OUT_FEAT), jnp.float32)],
    )(x, w2d, w_comp)


def _edge_body(xw_hbm, ei_hbm, et_hbm, out_hbm,
               src_pa, src_pb, dst_pa, dst_pb, typ_pa, typ_pb, gidx_pa, gidx_pb,
               rows_a, rows_b, dst_a, dst_b, hacc,
               sem_pa, sem_pb, sem_a, sem_b):
    cid = lax.axis_index("c")
    sid = lax.axis_index("s")
    wid = cid * NS + sid
    base = wid * EDGES_PER_WORKER

    def _prefetch(b, src_v, dst_v, typ_v, sem):
        off = base + b * SBLOCK
        pltpu.async_copy(ei_hbm.at[pl.ds(off, SBLOCK)], src_v, sem)
        pltpu.async_copy(ei_hbm.at[pl.ds(E + off, SBLOCK)], dst_v, sem)
        pltpu.async_copy(et_hbm.at[pl.ds(off, SBLOCK)], typ_v, sem)

    def _pwait(src_v, dst_v, typ_v, sem):
        pltpu.make_async_copy(ei_hbm.at[pl.ds(0, SBLOCK)], src_v, sem).wait()
        pltpu.make_async_copy(ei_hbm.at[pl.ds(0, SBLOCK)], dst_v, sem).wait()
        pltpu.make_async_copy(et_hbm.at[pl.ds(0, SBLOCK)], typ_v, sem).wait()

    _prefetch(0, src_pa, dst_pa, typ_pa, sem_pa)

    def _zrow(k, carry):
        i = k // (IN_FEAT // LANES)
        j = lax.rem(k, IN_FEAT // LANES)
        rows_a[i, pl.ds(j * LANES, LANES)] = jnp.zeros((LANES,), jnp.float32)
        return carry

    lax.fori_loop(0, CHUNK * (IN_FEAT // LANES), _zrow, None)
    for k in range(ROWS_PER_SUB // CHUNK):
        pltpu.sync_copy(rows_a, hacc.at[pl.ds(sid * ROWS_PER_SUB + k * CHUNK, CHUNK)])
    plsc.subcore_barrier()

    def _prep_start(gidx_v, dst_v, c, dst_small, rows, sem):
        for i in range(CHUNK // LANES):
            dst_small[pl.ds(i * LANES, LANES)] = dst_v[pl.ds(c * CHUNK + i * LANES, LANES)]
        return pltpu.async_copy(
            xw_hbm.at[gidx_v.at[pl.ds(c * CHUNK, CHUNK)]], rows, sem)

    def _wait(gidx_v, rows, sem):
        pltpu.make_async_copy(xw_hbm.at[gidx_v.at[pl.ds(0, CHUNK)]], rows, sem).wait()

    def _scatter(dst_small, rows):
        pltpu.sync_copy(rows, hacc.at[dst_small], add=True)

    def _gidx(gidx_v, src_v, typ_v):
        def body(i, c2):
            sl = pl.ds(i * LANES, LANES)
            gidx_v[sl] = src_v[sl] * NUM_RELS + typ_v[sl]
            return c2
        lax.fori_loop(0, SBLOCK // LANES, body, None)

    def _run_block(gidx_v, dst_v):
        _prep_start(gidx_v, dst_v, 0, dst_a, rows_a, sem_a)

        def _pipe(k, c2):
            c = 2 * k
            _prep_start(gidx_v, dst_v, c + 1, dst_b, rows_b, sem_b)
            _wait(gidx_v, rows_a, sem_a)
            _scatter(dst_a, rows_a)
            _prep_start(gidx_v, dst_v, c + 2, dst_a, rows_a, sem_a)
            _wait(gidx_v, rows_b, sem_b)
            _scatter(dst_b, rows_b)
            return c2

        lax.fori_loop(0, (CHUNKS_PER_SBLOCK - 1) // 2, _pipe, None)
        _wait(gidx_v, rows_a, sem_a)
        _scatter(dst_a, rows_a)

    def _block_pair(p, carry):
        b0 = 2 * p
        _pwait(src_pa, dst_pa, typ_pa, sem_pa)
        _prefetch(b0 + 1, src_pb, dst_pb, typ_pb, sem_pb)
        _gidx(gidx_pa, src_pa, typ_pa)
        _run_block(gidx_pa, dst_pa)

        _pwait(src_pb, dst_pb, typ_pb, sem_pb)
        _prefetch(b0 + 2, src_pa, dst_pa, typ_pa, sem_pa)
        _gidx(gidx_pb, src_pb, typ_pb)
        _run_block(gidx_pb, dst_pb)
        return carry

    lax.fori_loop(0, (NBLOCKS - 1) // 2, _block_pair, None)
    _pwait(src_pa, dst_pa, typ_pa, sem_pa)
    _gidx(gidx_pa, src_pa, typ_pa)
    _run_block(gidx_pa, dst_pa)
    plsc.subcore_barrier()

    pltpu.sync_copy(hacc.at[pl.ds(sid * ROWS_PER_SUB, ROWS_PER_SUB)],
                    out_hbm.at[cid, pl.ds(sid * ROWS_PER_SUB, ROWS_PER_SUB)])


def _edge_sc(xw_flat, ei_flat, edge_type):
    call = pl.kernel(
        _edge_body,
        out_type=jax.ShapeDtypeStruct((NC, ACC_ROWS, OUT_FEAT), jnp.float32),
        mesh=plsc.VectorSubcoreMesh(
            core_axis_name="c", subcore_axis_name="s",
            num_cores=NC, num_subcores=NS),
        scratch_types=[
            pltpu.VMEM((SBLOCK,), jnp.int32),
            pltpu.VMEM((SBLOCK,), jnp.int32),
            pltpu.VMEM((SBLOCK,), jnp.int32),
            pltpu.VMEM((SBLOCK,), jnp.int32),
            pltpu.VMEM((SBLOCK,), jnp.int32),
            pltpu.VMEM((SBLOCK,), jnp.int32),
            pltpu.VMEM((SBLOCK,), jnp.int32),
            pltpu.VMEM((SBLOCK,), jnp.int32),
            pltpu.VMEM((CHUNK, OUT_FEAT), jnp.float32),
            pltpu.VMEM((CHUNK, OUT_FEAT), jnp.float32),
            pltpu.VMEM((CHUNK,), jnp.int32),
            pltpu.VMEM((CHUNK,), jnp.int32),
            pltpu.VMEM_SHARED((ACC_ROWS, OUT_FEAT), jnp.float32),
            pltpu.SemaphoreType.DMA,
            pltpu.SemaphoreType.DMA,
            pltpu.SemaphoreType.DMA,
            pltpu.SemaphoreType.DMA,
        ],
    )
    return call(xw_flat, ei_flat, edge_type)


_CB = 2000


def _combine_body(p_ref, o_ref):
    o_ref[...] = p_ref[0] + p_ref[1]


def _combine(part):
    return pl.pallas_call(
        _combine_body,
        grid=(N // _CB,),
        in_specs=[pl.BlockSpec((NC, _CB, OUT_FEAT), lambda i: (0, i, 0))],
        out_specs=pl.BlockSpec((_CB, OUT_FEAT), lambda i: (i, 0)),
        out_shape=jax.ShapeDtypeStruct((N, OUT_FEAT), jnp.float32),
    )(part)


def kernel(x, edge_index, edge_type, weight, w_comp):
    w2d = weight.reshape(NUM_BASES * IN_FEAT, OUT_FEAT)
    xw = _xw_table(x, w2d, w_comp)
    xw_flat = xw.reshape(N * NUM_RELS, OUT_FEAT)
    ei_flat = edge_index.reshape(2 * E)
    part = _edge_sc(xw_flat, ei_flat, edge_type)
    return _combine(part)


# speedup vs baseline: 1.0885x; 1.0885x over previous
import functools

import jax
import jax.numpy as jnp
from jax import lax
from jax.experimental import pallas as pl
from jax.experimental.pallas import tpu as pltpu
from jax.experimental.pallas import tpu_sc as plsc

N = 10000
E = 320000
IN_FEAT = 128
OUT_FEAT = 128
NUM_RELS = 8
NUM_BASES = 4

NC = 2
NS = 16
NW = NC * NS
LANES = 16

EDGES_PER_WORKER = E // NW
CHUNK = 80
SBLOCK = 2000
CHUNKS_PER_SBLOCK = SBLOCK // CHUNK
NBLOCKS = EDGES_PER_WORKER // SBLOCK
ACC_ROWS = 10240
ROWS_PER_SUB = ACC_ROWS // NS


_BN = 1000


def _xw_body(x_ref, w2d_ref, wc_ref, out_ref, wbig_ref):
    @pl.when(pl.program_id(0) == 0)
    def _build_w():
        ki = lax.broadcasted_iota(jnp.int32, (NUM_RELS * IN_FEAT, NUM_BASES * IN_FEAT), 0)
        ji = lax.broadcasted_iota(jnp.int32, (NUM_RELS * IN_FEAT, NUM_BASES * IN_FEAT), 1)
        blk = (ki // NUM_RELS) == (ji // NUM_BASES)
        r_idx = lax.rem(ki, NUM_RELS)
        b_idx = lax.rem(ji, NUM_BASES)
        acc = jnp.zeros(ki.shape, jnp.float32)
        for r in range(NUM_RELS):
            for b in range(NUM_BASES):
                m = (r_idx == r) & (b_idx == b)
                acc = acc + jnp.where(m, wc_ref[r, b], 0.0)
        mmat = jnp.where(blk, acc, 0.0)
        wbig_ref[...] = jnp.dot(mmat, w2d_ref[...], preferred_element_type=jnp.float32)

    x = x_ref[...]
    for r in range(NUM_RELS):
        out_ref[:, OUT_FEAT * r:OUT_FEAT * (r + 1)] = jnp.dot(
            x, wbig_ref[IN_FEAT * r:IN_FEAT * (r + 1), :],
            preferred_element_type=jnp.float32)


def _xw_table(x, w2d, w_comp):
    return pl.pallas_call(
        _xw_body,
        grid=(N // _BN,),
        in_specs=[
            pl.BlockSpec((_BN, IN_FEAT), lambda i: (i, 0)),
            pl.BlockSpec((NUM_BASES * IN_FEAT, OUT_FEAT), lambda i: (0, 0)),
            pl.BlockSpec(memory_space=pltpu.SMEM),
        ],
        out_specs=pl.BlockSpec((_BN, NUM_RELS * OUT_FEAT), lambda i: (i, 0)),
        out_shape=jax.ShapeDtypeStruct((N, NUM_RELS * OUT_FEAT), jnp.float32),
        scratch_shapes=[pltpu.VMEM((NUM_RELS * IN_FEAT, OUT_FEAT), jnp.float32)],
    )(x, w2d, w_comp)


def _edge_body(xw_hbm, ei_hbm, et_hbm, out_hbm,
               src_pa, src_pb, dst_pa, dst_pb, typ_pa, typ_pb,
               rows_a, rows_b, rows_c, dst_a, dst_b, dst_c, hacc,
               sem_pa, sem_pb, sem_ga, sem_gb, sem_gc, sem_sa, sem_sb, sem_sc):
    cid = lax.axis_index("c")
    sid = lax.axis_index("s")
    wid = cid * NS + sid
    base = wid * EDGES_PER_WORKER

    def _prefetch(b, src_v, dst_v, typ_v, sem):
        off = base + b * SBLOCK
        pltpu.async_copy(ei_hbm.at[pl.ds(off, SBLOCK)], src_v, sem)
        pltpu.async_copy(ei_hbm.at[pl.ds(E + off, SBLOCK)], dst_v, sem)
        pltpu.async_copy(et_hbm.at[pl.ds(off, SBLOCK)], typ_v, sem)

    def _pwait(src_v, dst_v, typ_v, sem):
        pltpu.make_async_copy(ei_hbm.at[pl.ds(0, SBLOCK)], src_v, sem).wait()
        pltpu.make_async_copy(ei_hbm.at[pl.ds(0, SBLOCK)], dst_v, sem).wait()
        pltpu.make_async_copy(et_hbm.at[pl.ds(0, SBLOCK)], typ_v, sem).wait()

    _prefetch(0, src_pa, dst_pa, typ_pa, sem_pa)

    def _zrow(k, carry):
        i = k // (IN_FEAT // LANES)
        j = lax.rem(k, IN_FEAT // LANES)
        rows_a[i, pl.ds(j * LANES, LANES)] = jnp.zeros((LANES,), jnp.float32)
        return carry

    lax.fori_loop(0, CHUNK * (IN_FEAT // LANES), _zrow, None)
    for k in range(ROWS_PER_SUB // CHUNK):
        pltpu.sync_copy(rows_a, hacc.at[pl.ds(sid * ROWS_PER_SUB + k * CHUNK, CHUNK)])
    plsc.subcore_barrier()

    def _prep_g(gidx_v, dst_v, c, dst_small, rows, sem):
        for i in range(CHUNK // LANES):
            dst_small[pl.ds(i * LANES, LANES)] = dst_v[pl.ds(c * CHUNK + i * LANES, LANES)]
        pltpu.async_copy(
            xw_hbm.at[gidx_v.at[pl.ds(c * CHUNK, CHUNK)]], rows, sem)

    def _wait_g(gidx_v, rows, sem):
        pltpu.make_async_copy(xw_hbm.at[gidx_v.at[pl.ds(0, CHUNK)]], rows, sem).wait()

    def _start_s(dst_small, rows, sem):
        pltpu.async_copy(rows, hacc.at[dst_small], sem, add=True)

    def _wait_s(dst_small, rows, sem):
        pltpu.make_async_copy(rows, hacc.at[dst_small], sem).wait()

    def _gidx(src_v, typ_v):
        def body(i, c2):
            sl = pl.ds(i * LANES, LANES)
            src_v[sl] = src_v[sl] * NUM_RELS + typ_v[sl]
            return c2
        lax.fori_loop(0, SBLOCK // LANES, body, None)

    def _run_block(gidx_v, dst_v):
        bufs = [(dst_a, rows_a, sem_ga, sem_sa),
                (dst_b, rows_b, sem_gb, sem_sb),
                (dst_c, rows_c, sem_gc, sem_sc)]

        def g(c, i):
            ds, rw, sg, _ = bufs[i]
            _prep_g(gidx_v, dst_v, c, ds, rw, sg)

        def wg_s(c, i):
            ds, rw, sg, ss = bufs[i]
            _wait_g(gidx_v, rw, sg)
            _start_s(ds, rw, ss)

        def ws(i):
            ds, rw, _, ss = bufs[i]
            _wait_s(ds, rw, ss)

        g(0, 0)
        g(1, 1)
        wg_s(0, 0)
        g(2, 2)
        wg_s(1, 1)
        ws(0)
        g(3, 0)

        def _pipe(k, c2):
            c = 2 + 3 * k
            wg_s(c, 2)
            ws(1)
            g(c + 2, 1)
            wg_s(c + 1, 0)
            ws(2)
            g(c + 3, 2)
            wg_s(c + 2, 1)
            ws(0)
            g(c + 4, 0)
            return c2

        lax.fori_loop(0, (CHUNKS_PER_SBLOCK - 4) // 3, _pipe, None)

        wg_s(CHUNKS_PER_SBLOCK - 2, 2)
        wg_s(CHUNKS_PER_SBLOCK - 1, 0)
        ws(1)
        ws(2)
        ws(0)

    def _block_pair(p, carry):
        b0 = 2 * p
        _pwait(src_pa, dst_pa, typ_pa, sem_pa)
        _prefetch(b0 + 1, src_pb, dst_pb, typ_pb, sem_pb)
        _gidx(src_pa, typ_pa)
        _run_block(src_pa, dst_pa)

        _pwait(src_pb, dst_pb, typ_pb, sem_pb)
        _prefetch(b0 + 2, src_pa, dst_pa, typ_pa, sem_pa)
        _gidx(src_pb, typ_pb)
        _run_block(src_pb, dst_pb)
        return carry

    lax.fori_loop(0, (NBLOCKS - 1) // 2, _block_pair, None)
    _pwait(src_pa, dst_pa, typ_pa, sem_pa)
    _gidx(src_pa, typ_pa)
    _run_block(src_pa, dst_pa)
    plsc.subcore_barrier()

    pltpu.sync_copy(hacc.at[pl.ds(sid * ROWS_PER_SUB, ROWS_PER_SUB)],
                    out_hbm.at[cid, pl.ds(sid * ROWS_PER_SUB, ROWS_PER_SUB)])


def _edge_sc(xw_flat, ei_flat, edge_type):
    call = pl.kernel(
        _edge_body,
        out_type=jax.ShapeDtypeStruct((NC, ACC_ROWS, OUT_FEAT), jnp.float32),
        mesh=plsc.VectorSubcoreMesh(
            core_axis_name="c", subcore_axis_name="s",
            num_cores=NC, num_subcores=NS),
        scratch_types=[
            pltpu.VMEM((SBLOCK,), jnp.int32),
            pltpu.VMEM((SBLOCK,), jnp.int32),
            pltpu.VMEM((SBLOCK,), jnp.int32),
            pltpu.VMEM((SBLOCK,), jnp.int32),
            pltpu.VMEM((SBLOCK,), jnp.int32),
            pltpu.VMEM((SBLOCK,), jnp.int32),
            pltpu.VMEM((CHUNK, OUT_FEAT), jnp.float32),
            pltpu.VMEM((CHUNK, OUT_FEAT), jnp.float32),
            pltpu.VMEM((CHUNK, OUT_FEAT), jnp.float32),
            pltpu.VMEM((CHUNK,), jnp.int32),
            pltpu.VMEM((CHUNK,), jnp.int32),
            pltpu.VMEM((CHUNK,), jnp.int32),
            pltpu.VMEM_SHARED((ACC_ROWS, OUT_FEAT), jnp.float32),
            pltpu.SemaphoreType.DMA,
            pltpu.SemaphoreType.DMA,
            pltpu.SemaphoreType.DMA,
            pltpu.SemaphoreType.DMA,
            pltpu.SemaphoreType.DMA,
            pltpu.SemaphoreType.DMA,
            pltpu.SemaphoreType.DMA,
            pltpu.SemaphoreType.DMA,
        ],
    )
    return call(xw_flat, ei_flat, edge_type)


_CB = 2000


def _combine_body(p_ref, o_ref):
    o_ref[...] = p_ref[0] + p_ref[1]


def _combine(part):
    return pl.pallas_call(
        _combine_body,
        grid=(N // _CB,),
        in_specs=[pl.BlockSpec((NC, _CB, OUT_FEAT), lambda i: (0, i, 0))],
        out_specs=pl.BlockSpec((_CB, OUT_FEAT), lambda i: (i, 0)),
        out_shape=jax.ShapeDtypeStruct((N, OUT_FEAT), jnp.float32),
    )(part)


def kernel(x, edge_index, edge_type, weight, w_comp):
    w2d = weight.reshape(NUM_BASES * IN_FEAT, OUT_FEAT)
    xw = _xw_table(x, w2d, w_comp)
    xw_flat = xw.reshape(N * NUM_RELS, OUT_FEAT)
    ei_flat = edge_index.reshape(2 * E)
    part = _edge_sc(xw_flat, ei_flat, edge_type)
    return _combine(part)


# speedup vs baseline: 1.0887x; 1.0002x over previous
import functools

import jax
import jax.numpy as jnp
from jax import lax
from jax.experimental import pallas as pl
from jax.experimental.pallas import tpu as pltpu
from jax.experimental.pallas import tpu_sc as plsc

N = 10000
E = 320000
IN_FEAT = 128
OUT_FEAT = 128
NUM_RELS = 8
NUM_BASES = 4

NC = 2
NS = 16
NW = NC * NS
LANES = 16

EDGES_PER_WORKER = E // NW
CHUNK = 80
SBLOCK = 2000
CHUNKS_PER_SBLOCK = SBLOCK // CHUNK
NBLOCKS = EDGES_PER_WORKER // SBLOCK
ACC_ROWS = 10240
ROWS_PER_SUB = ACC_ROWS // NS


_BN = 1000


def _xw_body(x_ref, w2d_ref, wc_ref, out_ref, wbig_ref):
    @pl.when(pl.program_id(0) == 0)
    def _build_w():
        ki = lax.broadcasted_iota(jnp.int32, (NUM_RELS * IN_FEAT, NUM_BASES * IN_FEAT), 0)
        ji = lax.broadcasted_iota(jnp.int32, (NUM_RELS * IN_FEAT, NUM_BASES * IN_FEAT), 1)
        blk = (ki // NUM_RELS) == (ji // NUM_BASES)
        r_idx = lax.rem(ki, NUM_RELS)
        b_idx = lax.rem(ji, NUM_BASES)
        acc = jnp.zeros(ki.shape, jnp.float32)
        for r in range(NUM_RELS):
            for b in range(NUM_BASES):
                m = (r_idx == r) & (b_idx == b)
                acc = acc + jnp.where(m, wc_ref[r, b], 0.0)
        mmat = jnp.where(blk, acc, 0.0)
        wbig_ref[...] = jnp.dot(
            mmat, w2d_ref[...], preferred_element_type=jnp.float32
        ).astype(jnp.bfloat16)

    x = x_ref[...].astype(jnp.bfloat16)
    for r in range(NUM_RELS):
        out_ref[:, OUT_FEAT * r:OUT_FEAT * (r + 1)] = jnp.dot(
            x, wbig_ref[IN_FEAT * r:IN_FEAT * (r + 1), :],
            preferred_element_type=jnp.float32)


def _xw_table(x, w2d, w_comp):
    return pl.pallas_call(
        _xw_body,
        grid=(N // _BN,),
        in_specs=[
            pl.BlockSpec((_BN, IN_FEAT), lambda i: (i, 0)),
            pl.BlockSpec((NUM_BASES * IN_FEAT, OUT_FEAT), lambda i: (0, 0)),
            pl.BlockSpec(memory_space=pltpu.SMEM),
        ],
        out_specs=pl.BlockSpec((_BN, NUM_RELS * OUT_FEAT), lambda i: (i, 0)),
        out_shape=jax.ShapeDtypeStruct((N, NUM_RELS * OUT_FEAT), jnp.float32),
        scratch_shapes=[pltpu.VMEM((NUM_RELS * IN_FEAT, OUT_FEAT), jnp.bfloat16)],
    )(x, w2d, w_comp)


def _edge_body(xw_hbm, ei_hbm, et_hbm, out_hbm,
               src_pa, src_pb, dst_pa, dst_pb, typ_pa, typ_pb,
               rows_a, rows_b, rows_c, dst_a, dst_b, dst_c, hacc,
               sem_pa, sem_pb, sem_ga, sem_gb, sem_gc, sem_sa, sem_sb, sem_sc):
    cid = lax.axis_index("c")
    sid = lax.axis_index("s")
    wid = cid * NS + sid
    base = wid * EDGES_PER_WORKER

    def _prefetch(b, src_v, dst_v, typ_v, sem):
        off = base + b * SBLOCK
        pltpu.async_copy(ei_hbm.at[pl.ds(off, SBLOCK)], src_v, sem)
        pltpu.async_copy(ei_hbm.at[pl.ds(E + off, SBLOCK)], dst_v, sem)
        pltpu.async_copy(et_hbm.at[pl.ds(off, SBLOCK)], typ_v, sem)

    def _pwait(src_v, dst_v, typ_v, sem):
        pltpu.make_async_copy(ei_hbm.at[pl.ds(0, SBLOCK)], src_v, sem).wait()
        pltpu.make_async_copy(ei_hbm.at[pl.ds(0, SBLOCK)], dst_v, sem).wait()
        pltpu.make_async_copy(et_hbm.at[pl.ds(0, SBLOCK)], typ_v, sem).wait()

    _prefetch(0, src_pa, dst_pa, typ_pa, sem_pa)

    def _zrow(k, carry):
        i = k // (IN_FEAT // LANES)
        j = lax.rem(k, IN_FEAT // LANES)
        rows_a[i, pl.ds(j * LANES, LANES)] = jnp.zeros((LANES,), jnp.float32)
        return carry

    lax.fori_loop(0, CHUNK * (IN_FEAT // LANES), _zrow, None)
    for k in range(ROWS_PER_SUB // CHUNK):
        pltpu.async_copy(
            rows_a, hacc.at[pl.ds(sid * ROWS_PER_SUB + k * CHUNK, CHUNK)], sem_sa)
    for k in range(ROWS_PER_SUB // CHUNK):
        pltpu.make_async_copy(
            rows_a, hacc.at[pl.ds(sid * ROWS_PER_SUB + k * CHUNK, CHUNK)], sem_sa).wait()
    plsc.subcore_barrier()

    def _prep_g(gidx_v, dst_v, c, dst_small, rows, sem):
        for i in range(CHUNK // LANES):
            dst_small[pl.ds(i * LANES, LANES)] = dst_v[pl.ds(c * CHUNK + i * LANES, LANES)]
        pltpu.async_copy(
            xw_hbm.at[gidx_v.at[pl.ds(c * CHUNK, CHUNK)]], rows, sem)

    def _wait_g(gidx_v, rows, sem):
        pltpu.make_async_copy(xw_hbm.at[gidx_v.at[pl.ds(0, CHUNK)]], rows, sem).wait()

    def _start_s(dst_small, rows, sem):
        pltpu.async_copy(rows, hacc.at[dst_small], sem, add=True)

    def _wait_s(dst_small, rows, sem):
        pltpu.make_async_copy(rows, hacc.at[dst_small], sem).wait()

    def _gidx(src_v, typ_v):
        def body(i, c2):
            sl = pl.ds(i * LANES, LANES)
            src_v[sl] = src_v[sl] * NUM_RELS + typ_v[sl]
            return c2
        lax.fori_loop(0, SBLOCK // LANES, body, None)

    def _run_block(gidx_v, dst_v):
        bufs = [(dst_a, rows_a, sem_ga, sem_sa),
                (dst_b, rows_b, sem_gb, sem_sb),
                (dst_c, rows_c, sem_gc, sem_sc)]

        def g(c, i):
            ds, rw, sg, _ = bufs[i]
            _prep_g(gidx_v, dst_v, c, ds, rw, sg)

        def wg_s(c, i):
            ds, rw, sg, ss = bufs[i]
            _wait_g(gidx_v, rw, sg)
            _start_s(ds, rw, ss)

        def ws(i):
            ds, rw, _, ss = bufs[i]
            _wait_s(ds, rw, ss)

        g(0, 0)
        g(1, 1)
        wg_s(0, 0)
        g(2, 2)
        wg_s(1, 1)
        ws(0)
        g(3, 0)

        def _pipe(k, c2):
            c = 2 + 3 * k
            wg_s(c, 2)
            ws(1)
            g(c + 2, 1)
            wg_s(c + 1, 0)
            ws(2)
            g(c + 3, 2)
            wg_s(c + 2, 1)
            ws(0)
            g(c + 4, 0)
            return c2

        lax.fori_loop(0, (CHUNKS_PER_SBLOCK - 4) // 3, _pipe, None)

        wg_s(CHUNKS_PER_SBLOCK - 2, 2)
        wg_s(CHUNKS_PER_SBLOCK - 1, 0)
        ws(1)
        ws(2)
        ws(0)

    def _block_pair(p, carry):
        b0 = 2 * p
        _pwait(src_pa, dst_pa, typ_pa, sem_pa)
        _prefetch(b0 + 1, src_pb, dst_pb, typ_pb, sem_pb)
        _gidx(src_pa, typ_pa)
        _run_block(src_pa, dst_pa)

        _pwait(src_pb, dst_pb, typ_pb, sem_pb)
        _prefetch(b0 + 2, src_pa, dst_pa, typ_pa, sem_pa)
        _gidx(src_pb, typ_pb)
        _run_block(src_pb, dst_pb)
        return carry

    lax.fori_loop(0, (NBLOCKS - 1) // 2, _block_pair, None)
    _pwait(src_pa, dst_pa, typ_pa, sem_pa)
    _gidx(src_pa, typ_pa)
    _run_block(src_pa, dst_pa)
    plsc.subcore_barrier()

    pltpu.sync_copy(hacc.at[pl.ds(sid * ROWS_PER_SUB, ROWS_PER_SUB)],
                    out_hbm.at[cid, pl.ds(sid * ROWS_PER_SUB, ROWS_PER_SUB)])


def _edge_sc(xw_flat, ei_flat, edge_type):
    call = pl.kernel(
        _edge_body,
        out_type=jax.ShapeDtypeStruct((NC, ACC_ROWS, OUT_FEAT), jnp.float32),
        mesh=plsc.VectorSubcoreMesh(
            core_axis_name="c", subcore_axis_name="s",
            num_cores=NC, num_subcores=NS),
        scratch_types=[
            pltpu.VMEM((SBLOCK,), jnp.int32),
            pltpu.VMEM((SBLOCK,), jnp.int32),
            pltpu.VMEM((SBLOCK,), jnp.int32),
            pltpu.VMEM((SBLOCK,), jnp.int32),
            pltpu.VMEM((SBLOCK,), jnp.int32),
            pltpu.VMEM((SBLOCK,), jnp.int32),
            pltpu.VMEM((CHUNK, OUT_FEAT), jnp.float32),
            pltpu.VMEM((CHUNK, OUT_FEAT), jnp.float32),
            pltpu.VMEM((CHUNK, OUT_FEAT), jnp.float32),
            pltpu.VMEM((CHUNK,), jnp.int32),
            pltpu.VMEM((CHUNK,), jnp.int32),
            pltpu.VMEM((CHUNK,), jnp.int32),
            pltpu.VMEM_SHARED((ACC_ROWS, OUT_FEAT), jnp.float32),
            pltpu.SemaphoreType.DMA,
            pltpu.SemaphoreType.DMA,
            pltpu.SemaphoreType.DMA,
            pltpu.SemaphoreType.DMA,
            pltpu.SemaphoreType.DMA,
            pltpu.SemaphoreType.DMA,
            pltpu.SemaphoreType.DMA,
            pltpu.SemaphoreType.DMA,
        ],
    )
    return call(xw_flat, ei_flat, edge_type)


_CB = 2000


def _combine_body(p_ref, o_ref):
    o_ref[...] = p_ref[0] + p_ref[1]


def _combine(part):
    return pl.pallas_call(
        _combine_body,
        grid=(N // _CB,),
        in_specs=[pl.BlockSpec((NC, _CB, OUT_FEAT), lambda i: (0, i, 0))],
        out_specs=pl.BlockSpec((_CB, OUT_FEAT), lambda i: (i, 0)),
        out_shape=jax.ShapeDtypeStruct((N, OUT_FEAT), jnp.float32),
    )(part)


def kernel(x, edge_index, edge_type, weight, w_comp):
    w2d = weight.reshape(NUM_BASES * IN_FEAT, OUT_FEAT)
    xw = _xw_table(x, w2d, w_comp)
    xw_flat = xw.reshape(N * NUM_RELS, OUT_FEAT)
    ei_flat = edge_index.reshape(2 * E)
    part = _edge_sc(xw_flat, ei_flat, edge_type)
    return _combine(part)


# speedup vs baseline: 1.3650x; 1.2538x over previous
import functools

import jax
import jax.numpy as jnp
from jax import lax
from jax.experimental import pallas as pl
from jax.experimental.pallas import tpu as pltpu
from jax.experimental.pallas import tpu_sc as plsc

N = 10000
E = 320000
IN_FEAT = 128
OUT_FEAT = 128
NUM_RELS = 8
NUM_BASES = 4

NC = 2
NS = 16
NW = NC * NS
LANES = 16

EDGES_PER_WORKER = E // NW
CHUNK = 80
SBLOCK = 2000
CHUNKS_PER_SBLOCK = SBLOCK // CHUNK
NBLOCKS = EDGES_PER_WORKER // SBLOCK
ACC_ROWS = 10240
ROWS_PER_SUB = ACC_ROWS // NS


_BN = 1000


def _xw_body(x_ref, w2d_ref, wc_ref, out_ref, wbig_ref):
    @pl.when(pl.program_id(0) == 0)
    def _build_w():
        ki = lax.broadcasted_iota(jnp.int32, (NUM_RELS * IN_FEAT, NUM_BASES * IN_FEAT), 0)
        ji = lax.broadcasted_iota(jnp.int32, (NUM_RELS * IN_FEAT, NUM_BASES * IN_FEAT), 1)
        blk = (ki // NUM_RELS) == (ji // NUM_BASES)
        r_idx = lax.rem(ki, NUM_RELS)
        b_idx = lax.rem(ji, NUM_BASES)
        acc = jnp.zeros(ki.shape, jnp.float32)
        for r in range(NUM_RELS):
            for b in range(NUM_BASES):
                m = (r_idx == r) & (b_idx == b)
                acc = acc + jnp.where(m, wc_ref[r, b], 0.0)
        mmat = jnp.where(blk, acc, 0.0)
        wbig_ref[...] = jnp.dot(
            mmat, w2d_ref[...], preferred_element_type=jnp.float32
        ).astype(jnp.bfloat16)

    x = x_ref[...].astype(jnp.bfloat16)
    for r in range(NUM_RELS):
        out_ref[r] = jnp.dot(
            x, wbig_ref[IN_FEAT * r:IN_FEAT * (r + 1), :],
            preferred_element_type=jnp.float32)


def _xw_table(x, w2d, w_comp):
    return pl.pallas_call(
        _xw_body,
        grid=(N // _BN,),
        in_specs=[
            pl.BlockSpec((_BN, IN_FEAT), lambda i: (i, 0)),
            pl.BlockSpec((NUM_BASES * IN_FEAT, OUT_FEAT), lambda i: (0, 0)),
            pl.BlockSpec(memory_space=pltpu.SMEM),
        ],
        out_specs=pl.BlockSpec((NUM_RELS, _BN, OUT_FEAT), lambda i: (0, i, 0)),
        out_shape=jax.ShapeDtypeStruct((NUM_RELS, N, OUT_FEAT), jnp.float32),
        scratch_shapes=[pltpu.VMEM((NUM_RELS * IN_FEAT, OUT_FEAT), jnp.bfloat16)],
    )(x, w2d, w_comp)


def _edge_body(xw_hbm, ei_hbm, et_hbm, out_hbm,
               src_pa, src_pb, dst_pa, dst_pb, typ_pa, typ_pb,
               rows_a, rows_b, rows_c, dst_a, dst_b, dst_c, hacc,
               sem_pa, sem_pb, sem_ga, sem_gb, sem_gc, sem_sa, sem_sb, sem_sc):
    cid = lax.axis_index("c")
    sid = lax.axis_index("s")
    wid = cid * NS + sid
    base = wid * EDGES_PER_WORKER

    def _prefetch(b, src_v, dst_v, typ_v, sem):
        off = base + b * SBLOCK
        pltpu.async_copy(ei_hbm.at[pl.ds(off, SBLOCK)], src_v, sem)
        pltpu.async_copy(ei_hbm.at[pl.ds(E + off, SBLOCK)], dst_v, sem)
        pltpu.async_copy(et_hbm.at[pl.ds(off, SBLOCK)], typ_v, sem)

    def _pwait(src_v, dst_v, typ_v, sem):
        pltpu.make_async_copy(ei_hbm.at[pl.ds(0, SBLOCK)], src_v, sem).wait()
        pltpu.make_async_copy(ei_hbm.at[pl.ds(0, SBLOCK)], dst_v, sem).wait()
        pltpu.make_async_copy(et_hbm.at[pl.ds(0, SBLOCK)], typ_v, sem).wait()

    _prefetch(0, src_pa, dst_pa, typ_pa, sem_pa)

    def _zrow(k, carry):
        i = k // (IN_FEAT // LANES)
        j = lax.rem(k, IN_FEAT // LANES)
        rows_a[i, pl.ds(j * LANES, LANES)] = jnp.zeros((LANES,), jnp.float32)
        return carry

    lax.fori_loop(0, CHUNK * (IN_FEAT // LANES), _zrow, None)
    for k in range(ROWS_PER_SUB // CHUNK):
        pltpu.async_copy(
            rows_a, hacc.at[pl.ds(sid * ROWS_PER_SUB + k * CHUNK, CHUNK)], sem_sa)
    for k in range(ROWS_PER_SUB // CHUNK):
        pltpu.make_async_copy(
            rows_a, hacc.at[pl.ds(sid * ROWS_PER_SUB + k * CHUNK, CHUNK)], sem_sa).wait()
    plsc.subcore_barrier()

    def _prep_g(gidx_v, dst_v, c, dst_small, rows, sem):
        for i in range(CHUNK // LANES):
            dst_small[pl.ds(i * LANES, LANES)] = dst_v[pl.ds(c * CHUNK + i * LANES, LANES)]
        pltpu.async_copy(
            xw_hbm.at[gidx_v.at[pl.ds(c * CHUNK, CHUNK)]], rows, sem)

    def _wait_g(gidx_v, rows, sem):
        pltpu.make_async_copy(xw_hbm.at[gidx_v.at[pl.ds(0, CHUNK)]], rows, sem).wait()

    def _start_s(dst_small, rows, sem):
        pltpu.async_copy(rows, hacc.at[dst_small], sem, add=True)

    def _wait_s(dst_small, rows, sem):
        pltpu.make_async_copy(rows, hacc.at[dst_small], sem).wait()

    def _gidx(src_v, typ_v):
        def body(i, c2):
            sl = pl.ds(i * LANES, LANES)
            src_v[sl] = typ_v[sl] * N + src_v[sl]
            return c2
        lax.fori_loop(0, SBLOCK // LANES, body, None)

    def _run_block(gidx_v, dst_v):
        bufs = [(dst_a, rows_a, sem_ga, sem_sa),
                (dst_b, rows_b, sem_gb, sem_sb),
                (dst_c, rows_c, sem_gc, sem_sc)]

        def g(c, i):
            ds, rw, sg, _ = bufs[i]
            _prep_g(gidx_v, dst_v, c, ds, rw, sg)

        def wg_s(c, i):
            ds, rw, sg, ss = bufs[i]
            _wait_g(gidx_v, rw, sg)
            _start_s(ds, rw, ss)

        def ws(i):
            ds, rw, _, ss = bufs[i]
            _wait_s(ds, rw, ss)

        g(0, 0)
        g(1, 1)
        wg_s(0, 0)
        g(2, 2)
        wg_s(1, 1)
        ws(0)
        g(3, 0)

        def _pipe(k, c2):
            c = 2 + 3 * k
            wg_s(c, 2)
            ws(1)
            g(c + 2, 1)
            wg_s(c + 1, 0)
            ws(2)
            g(c + 3, 2)
            wg_s(c + 2, 1)
            ws(0)
            g(c + 4, 0)
            return c2

        lax.fori_loop(0, (CHUNKS_PER_SBLOCK - 4) // 3, _pipe, None)

        wg_s(CHUNKS_PER_SBLOCK - 2, 2)
        wg_s(CHUNKS_PER_SBLOCK - 1, 0)
        ws(1)
        ws(2)
        ws(0)

    def _block_pair(p, carry):
        b0 = 2 * p
        _pwait(src_pa, dst_pa, typ_pa, sem_pa)
        _prefetch(b0 + 1, src_pb, dst_pb, typ_pb, sem_pb)
        _gidx(src_pa, typ_pa)
        _run_block(src_pa, dst_pa)

        _pwait(src_pb, dst_pb, typ_pb, sem_pb)
        _prefetch(b0 + 2, src_pa, dst_pa, typ_pa, sem_pa)
        _gidx(src_pb, typ_pb)
        _run_block(src_pb, dst_pb)
        return carry

    lax.fori_loop(0, (NBLOCKS - 1) // 2, _block_pair, None)
    _pwait(src_pa, dst_pa, typ_pa, sem_pa)
    _gidx(src_pa, typ_pa)
    _run_block(src_pa, dst_pa)
    plsc.subcore_barrier()

    pltpu.sync_copy(hacc.at[pl.ds(sid * ROWS_PER_SUB, ROWS_PER_SUB)],
                    out_hbm.at[cid, pl.ds(sid * ROWS_PER_SUB, ROWS_PER_SUB)])


def _edge_sc(xw_flat, ei_flat, edge_type):
    call = pl.kernel(
        _edge_body,
        out_type=jax.ShapeDtypeStruct((NC, ACC_ROWS, OUT_FEAT), jnp.float32),
        mesh=plsc.VectorSubcoreMesh(
            core_axis_name="c", subcore_axis_name="s",
            num_cores=NC, num_subcores=NS),
        scratch_types=[
            pltpu.VMEM((SBLOCK,), jnp.int32),
            pltpu.VMEM((SBLOCK,), jnp.int32),
            pltpu.VMEM((SBLOCK,), jnp.int32),
            pltpu.VMEM((SBLOCK,), jnp.int32),
            pltpu.VMEM((SBLOCK,), jnp.int32),
            pltpu.VMEM((SBLOCK,), jnp.int32),
            pltpu.VMEM((CHUNK, OUT_FEAT), jnp.float32),
            pltpu.VMEM((CHUNK, OUT_FEAT), jnp.float32),
            pltpu.VMEM((CHUNK, OUT_FEAT), jnp.float32),
            pltpu.VMEM((CHUNK,), jnp.int32),
            pltpu.VMEM((CHUNK,), jnp.int32),
            pltpu.VMEM((CHUNK,), jnp.int32),
            pltpu.VMEM_SHARED((ACC_ROWS, OUT_FEAT), jnp.float32),
            pltpu.SemaphoreType.DMA,
            pltpu.SemaphoreType.DMA,
            pltpu.SemaphoreType.DMA,
            pltpu.SemaphoreType.DMA,
            pltpu.SemaphoreType.DMA,
            pltpu.SemaphoreType.DMA,
            pltpu.SemaphoreType.DMA,
            pltpu.SemaphoreType.DMA,
        ],
    )
    return call(xw_flat, ei_flat, edge_type)


_CB = 2000


def _combine_body(p_ref, o_ref):
    o_ref[...] = p_ref[0] + p_ref[1]


def _combine(part):
    return pl.pallas_call(
        _combine_body,
        grid=(N // _CB,),
        in_specs=[pl.BlockSpec((NC, _CB, OUT_FEAT), lambda i: (0, i, 0))],
        out_specs=pl.BlockSpec((_CB, OUT_FEAT), lambda i: (i, 0)),
        out_shape=jax.ShapeDtypeStruct((N, OUT_FEAT), jnp.float32),
    )(part)


def kernel(x, edge_index, edge_type, weight, w_comp):
    w2d = weight.reshape(NUM_BASES * IN_FEAT, OUT_FEAT)
    xw = _xw_table(x, w2d, w_comp)
    xw_flat = xw.reshape(NUM_RELS * N, OUT_FEAT)
    ei_flat = edge_index.reshape(2 * E)
    part = _edge_sc(xw_flat, ei_flat, edge_type)
    return _combine(part)


# speedup vs baseline: 1.4041x; 1.0286x over previous
import functools

import jax
import jax.numpy as jnp
from jax import lax
from jax.experimental import pallas as pl
from jax.experimental.pallas import tpu as pltpu
from jax.experimental.pallas import tpu_sc as plsc

N = 10000
E = 320000
IN_FEAT = 128
OUT_FEAT = 128
NUM_RELS = 8
NUM_BASES = 4

NC = 2
NS = 16
NW = NC * NS
LANES = 16

EDGES_PER_WORKER = E // NW
CHUNK = 80
SBLOCK = 2000
CHUNKS_PER_SBLOCK = SBLOCK // CHUNK
NBLOCKS = EDGES_PER_WORKER // SBLOCK
ACC_ROWS = 10240
ROWS_PER_SUB = ACC_ROWS // NS


_BN = 1000


def _xw_body(x_ref, w2d_ref, wc_ref, out_ref, wbig_ref):
    @pl.when(pl.program_id(0) == 0)
    def _build_w():
        KK, JJ = NUM_RELS * IN_FEAT, NUM_BASES * IN_FEAT
        ki = lax.broadcasted_iota(jnp.int32, (KK, JJ), 0)
        ji = lax.broadcasted_iota(jnp.int32, (KK, JJ), 1)
        eq = ((ki // NUM_RELS) == (ji // NUM_BASES)).astype(jnp.float32)
        ar = lax.broadcasted_iota(jnp.int32, (KK, NUM_RELS), 0)
        ac = lax.broadcasted_iota(jnp.int32, (KK, NUM_RELS), 1)
        amat = (lax.rem(ar, NUM_RELS) == ac).astype(jnp.float32)
        wr = lax.broadcasted_iota(jnp.int32, (NUM_RELS, JJ), 0)
        wj = lax.broadcasted_iota(jnp.int32, (NUM_RELS, JJ), 1)
        wjb = lax.rem(wj, NUM_BASES)
        wcb = jnp.zeros((NUM_RELS, JJ), jnp.float32)
        for r in range(NUM_RELS):
            for b in range(NUM_BASES):
                wcb = wcb + jnp.where((wr == r) & (wjb == b), wc_ref[r, b], 0.0)
        tiled = jnp.dot(amat, wcb, preferred_element_type=jnp.float32)
        mmat = eq * tiled
        wbig_ref[...] = jnp.dot(
            mmat, w2d_ref[...], preferred_element_type=jnp.float32
        ).astype(jnp.bfloat16)

    x = x_ref[...].astype(jnp.bfloat16)
    for r in range(NUM_RELS):
        out_ref[r] = jnp.dot(
            x, wbig_ref[IN_FEAT * r:IN_FEAT * (r + 1), :],
            preferred_element_type=jnp.float32)


def _xw_table(x, w2d, w_comp):
    return pl.pallas_call(
        _xw_body,
        grid=(N // _BN,),
        in_specs=[
            pl.BlockSpec((_BN, IN_FEAT), lambda i: (i, 0)),
            pl.BlockSpec((NUM_BASES * IN_FEAT, OUT_FEAT), lambda i: (0, 0)),
            pl.BlockSpec(memory_space=pltpu.SMEM),
        ],
        out_specs=pl.BlockSpec((NUM_RELS, _BN, OUT_FEAT), lambda i: (0, i, 0)),
        out_shape=jax.ShapeDtypeStruct((NUM_RELS, N, OUT_FEAT), jnp.float32),
        scratch_shapes=[pltpu.VMEM((NUM_RELS * IN_FEAT, OUT_FEAT), jnp.bfloat16)],
    )(x, w2d, w_comp)


def _edge_body(xw_hbm, ei_hbm, et_hbm, out_hbm,
               src_pa, src_pb, dst_pa, dst_pb, typ_pa, typ_pb,
               rows_a, rows_b, rows_c, dst_a, dst_b, dst_c, hacc,
               sem_pa, sem_pb, sem_ga, sem_gb, sem_gc, sem_sa, sem_sb, sem_sc):
    cid = lax.axis_index("c")
    sid = lax.axis_index("s")
    wid = cid * NS + sid
    base = wid * EDGES_PER_WORKER

    def _prefetch(b, src_v, dst_v, typ_v, sem):
        off = base + b * SBLOCK
        pltpu.async_copy(ei_hbm.at[pl.ds(off, SBLOCK)], src_v, sem)
        pltpu.async_copy(ei_hbm.at[pl.ds(E + off, SBLOCK)], dst_v, sem)
        pltpu.async_copy(et_hbm.at[pl.ds(off, SBLOCK)], typ_v, sem)

    def _pwait(src_v, dst_v, typ_v, sem):
        pltpu.make_async_copy(ei_hbm.at[pl.ds(0, SBLOCK)], src_v, sem).wait()
        pltpu.make_async_copy(ei_hbm.at[pl.ds(0, SBLOCK)], dst_v, sem).wait()
        pltpu.make_async_copy(et_hbm.at[pl.ds(0, SBLOCK)], typ_v, sem).wait()

    _prefetch(0, src_pa, dst_pa, typ_pa, sem_pa)

    def _zrow(k, carry):
        i = k // (IN_FEAT // LANES)
        j = lax.rem(k, IN_FEAT // LANES)
        rows_a[i, pl.ds(j * LANES, LANES)] = jnp.zeros((LANES,), jnp.float32)
        return carry

    lax.fori_loop(0, CHUNK * (IN_FEAT // LANES), _zrow, None)
    for k in range(ROWS_PER_SUB // CHUNK):
        pltpu.async_copy(
            rows_a, hacc.at[pl.ds(sid * ROWS_PER_SUB + k * CHUNK, CHUNK)], sem_sa)
    for k in range(ROWS_PER_SUB // CHUNK):
        pltpu.make_async_copy(
            rows_a, hacc.at[pl.ds(sid * ROWS_PER_SUB + k * CHUNK, CHUNK)], sem_sa).wait()
    plsc.subcore_barrier()

    def _prep_g(gidx_v, dst_v, c, dst_small, rows, sem):
        for i in range(CHUNK // LANES):
            dst_small[pl.ds(i * LANES, LANES)] = dst_v[pl.ds(c * CHUNK + i * LANES, LANES)]
        pltpu.async_copy(
            xw_hbm.at[gidx_v.at[pl.ds(c * CHUNK, CHUNK)]], rows, sem)

    def _wait_g(gidx_v, rows, sem):
        pltpu.make_async_copy(xw_hbm.at[gidx_v.at[pl.ds(0, CHUNK)]], rows, sem).wait()

    def _start_s(dst_small, rows, sem):
        pltpu.async_copy(rows, hacc.at[dst_small], sem, add=True)

    def _wait_s(dst_small, rows, sem):
        pltpu.make_async_copy(rows, hacc.at[dst_small], sem).wait()

    def _gidx(src_v, typ_v):
        def body(i, c2):
            sl = pl.ds(i * LANES, LANES)
            src_v[sl] = typ_v[sl] * N + src_v[sl]
            return c2
        lax.fori_loop(0, SBLOCK // LANES, body, None)

    def _run_block(gidx_v, dst_v):
        bufs = [(dst_a, rows_a, sem_ga, sem_sa),
                (dst_b, rows_b, sem_gb, sem_sb),
                (dst_c, rows_c, sem_gc, sem_sc)]

        def g(c, i):
            ds, rw, sg, _ = bufs[i]
            _prep_g(gidx_v, dst_v, c, ds, rw, sg)

        def wg_s(c, i):
            ds, rw, sg, ss = bufs[i]
            _wait_g(gidx_v, rw, sg)
            _start_s(ds, rw, ss)

        def ws(i):
            ds, rw, _, ss = bufs[i]
            _wait_s(ds, rw, ss)

        g(0, 0)
        g(1, 1)
        wg_s(0, 0)
        g(2, 2)
        wg_s(1, 1)
        ws(0)
        g(3, 0)

        def _pipe(k, c2):
            c = 2 + 3 * k
            wg_s(c, 2)
            ws(1)
            g(c + 2, 1)
            wg_s(c + 1, 0)
            ws(2)
            g(c + 3, 2)
            wg_s(c + 2, 1)
            ws(0)
            g(c + 4, 0)
            return c2

        lax.fori_loop(0, (CHUNKS_PER_SBLOCK - 4) // 3, _pipe, None)

        wg_s(CHUNKS_PER_SBLOCK - 2, 2)
        wg_s(CHUNKS_PER_SBLOCK - 1, 0)
        ws(1)
        ws(2)
        ws(0)

    def _block_pair(p, carry):
        b0 = 2 * p
        _pwait(src_pa, dst_pa, typ_pa, sem_pa)
        _prefetch(b0 + 1, src_pb, dst_pb, typ_pb, sem_pb)
        _gidx(src_pa, typ_pa)
        _run_block(src_pa, dst_pa)

        _pwait(src_pb, dst_pb, typ_pb, sem_pb)
        _prefetch(b0 + 2, src_pa, dst_pa, typ_pa, sem_pa)
        _gidx(src_pb, typ_pb)
        _run_block(src_pb, dst_pb)
        return carry

    lax.fori_loop(0, (NBLOCKS - 1) // 2, _block_pair, None)
    _pwait(src_pa, dst_pa, typ_pa, sem_pa)
    _gidx(src_pa, typ_pa)
    _run_block(src_pa, dst_pa)
    plsc.subcore_barrier()

    pltpu.sync_copy(hacc.at[pl.ds(sid * ROWS_PER_SUB, ROWS_PER_SUB)],
                    out_hbm.at[cid, pl.ds(sid * ROWS_PER_SUB, ROWS_PER_SUB)])


def _edge_sc(xw_flat, ei_flat, edge_type):
    call = pl.kernel(
        _edge_body,
        out_type=jax.ShapeDtypeStruct((NC, ACC_ROWS, OUT_FEAT), jnp.float32),
        mesh=plsc.VectorSubcoreMesh(
            core_axis_name="c", subcore_axis_name="s",
            num_cores=NC, num_subcores=NS),
        scratch_types=[
            pltpu.VMEM((SBLOCK,), jnp.int32),
            pltpu.VMEM((SBLOCK,), jnp.int32),
            pltpu.VMEM((SBLOCK,), jnp.int32),
            pltpu.VMEM((SBLOCK,), jnp.int32),
            pltpu.VMEM((SBLOCK,), jnp.int32),
            pltpu.VMEM((SBLOCK,), jnp.int32),
            pltpu.VMEM((CHUNK, OUT_FEAT), jnp.float32),
            pltpu.VMEM((CHUNK, OUT_FEAT), jnp.float32),
            pltpu.VMEM((CHUNK, OUT_FEAT), jnp.float32),
            pltpu.VMEM((CHUNK,), jnp.int32),
            pltpu.VMEM((CHUNK,), jnp.int32),
            pltpu.VMEM((CHUNK,), jnp.int32),
            pltpu.VMEM_SHARED((ACC_ROWS, OUT_FEAT), jnp.float32),
            pltpu.SemaphoreType.DMA,
            pltpu.SemaphoreType.DMA,
            pltpu.SemaphoreType.DMA,
            pltpu.SemaphoreType.DMA,
            pltpu.SemaphoreType.DMA,
            pltpu.SemaphoreType.DMA,
            pltpu.SemaphoreType.DMA,
            pltpu.SemaphoreType.DMA,
        ],
    )
    return call(xw_flat, ei_flat, edge_type)


_CB = 2000


def _combine_body(p_ref, o_ref):
    o_ref[...] = p_ref[0] + p_ref[1]


def _combine(part):
    return pl.pallas_call(
        _combine_body,
        grid=(N // _CB,),
        in_specs=[pl.BlockSpec((NC, _CB, OUT_FEAT), lambda i: (0, i, 0))],
        out_specs=pl.BlockSpec((_CB, OUT_FEAT), lambda i: (i, 0)),
        out_shape=jax.ShapeDtypeStruct((N, OUT_FEAT), jnp.float32),
    )(part)


def kernel(x, edge_index, edge_type, weight, w_comp):
    w2d = weight.reshape(NUM_BASES * IN_FEAT, OUT_FEAT)
    xw = _xw_table(x, w2d, w_comp)
    xw_flat = xw.reshape(NUM_RELS * N, OUT_FEAT)
    ei_flat = edge_index.reshape(2 * E)
    part = _edge_sc(xw_flat, ei_flat, edge_type)
    return _combine(part)


# speedup vs baseline: 1.4148x; 1.0076x over previous
import functools

import jax
import jax.numpy as jnp
from jax import lax
from jax.experimental import pallas as pl
from jax.experimental.pallas import tpu as pltpu
from jax.experimental.pallas import tpu_sc as plsc

N = 10000
E = 320000
IN_FEAT = 128
OUT_FEAT = 128
NUM_RELS = 8
NUM_BASES = 4

NC = 2
NS = 16
NW = NC * NS
LANES = 16

EDGES_PER_WORKER = E // NW
CHUNK = 80
SBLOCK = 2000
CHUNKS_PER_SBLOCK = SBLOCK // CHUNK
NBLOCKS = EDGES_PER_WORKER // SBLOCK
ACC_ROWS = 10240
ROWS_PER_SUB = ACC_ROWS // NS


_BN = 2000


def _xw_body(x_ref, w2d_ref, wc_ref, out_ref, wbig_ref):
    @pl.when(pl.program_id(0) == 0)
    def _build_w():
        KK, JJ = NUM_RELS * IN_FEAT, NUM_BASES * IN_FEAT
        ki = lax.broadcasted_iota(jnp.int32, (KK, JJ), 0)
        ji = lax.broadcasted_iota(jnp.int32, (KK, JJ), 1)
        eq = ((ki // NUM_RELS) == (ji // NUM_BASES)).astype(jnp.float32)
        ar = lax.broadcasted_iota(jnp.int32, (KK, NUM_RELS), 0)
        ac = lax.broadcasted_iota(jnp.int32, (KK, NUM_RELS), 1)
        amat = (lax.rem(ar, NUM_RELS) == ac).astype(jnp.float32)
        wr = lax.broadcasted_iota(jnp.int32, (NUM_RELS, JJ), 0)
        wj = lax.broadcasted_iota(jnp.int32, (NUM_RELS, JJ), 1)
        wjb = lax.rem(wj, NUM_BASES)
        wcb = jnp.zeros((NUM_RELS, JJ), jnp.float32)
        for r in range(NUM_RELS):
            for b in range(NUM_BASES):
                wcb = wcb + jnp.where((wr == r) & (wjb == b), wc_ref[r, b], 0.0)
        tiled = jnp.dot(amat, wcb, preferred_element_type=jnp.float32)
        mmat = eq * tiled
        wbig_ref[...] = jnp.dot(
            mmat, w2d_ref[...], preferred_element_type=jnp.float32
        ).astype(jnp.bfloat16)

    x = x_ref[...].astype(jnp.bfloat16)
    for r in range(NUM_RELS):
        out_ref[r] = jnp.dot(
            x, wbig_ref[IN_FEAT * r:IN_FEAT * (r + 1), :],
            preferred_element_type=jnp.float32)


def _xw_table(x, w2d, w_comp):
    return pl.pallas_call(
        _xw_body,
        grid=(N // _BN,),
        in_specs=[
            pl.BlockSpec((_BN, IN_FEAT), lambda i: (i, 0)),
            pl.BlockSpec((NUM_BASES * IN_FEAT, OUT_FEAT), lambda i: (0, 0)),
            pl.BlockSpec(memory_space=pltpu.SMEM),
        ],
        out_specs=pl.BlockSpec((NUM_RELS, _BN, OUT_FEAT), lambda i: (0, i, 0)),
        out_shape=jax.ShapeDtypeStruct((NUM_RELS, N, OUT_FEAT), jnp.float32),
        scratch_shapes=[pltpu.VMEM((NUM_RELS * IN_FEAT, OUT_FEAT), jnp.bfloat16)],
    )(x, w2d, w_comp)


def _edge_body(xw_hbm, ei_hbm, et_hbm, out_hbm,
               src_pa, src_pb, dst_pa, dst_pb, typ_pa, typ_pb,
               rows_a, rows_b, rows_c, dst_a, dst_b, dst_c, hacc,
               sem_pa, sem_pb, sem_ga, sem_gb, sem_gc, sem_sa, sem_sb, sem_sc):
    cid = lax.axis_index("c")
    sid = lax.axis_index("s")
    wid = cid * NS + sid
    base = wid * EDGES_PER_WORKER

    def _prefetch(b, src_v, dst_v, typ_v, sem):
        off = base + b * SBLOCK
        pltpu.async_copy(ei_hbm.at[pl.ds(off, SBLOCK)], src_v, sem)
        pltpu.async_copy(ei_hbm.at[pl.ds(E + off, SBLOCK)], dst_v, sem)
        pltpu.async_copy(et_hbm.at[pl.ds(off, SBLOCK)], typ_v, sem)

    def _pwait(src_v, dst_v, typ_v, sem):
        pltpu.make_async_copy(ei_hbm.at[pl.ds(0, SBLOCK)], src_v, sem).wait()
        pltpu.make_async_copy(ei_hbm.at[pl.ds(0, SBLOCK)], dst_v, sem).wait()
        pltpu.make_async_copy(et_hbm.at[pl.ds(0, SBLOCK)], typ_v, sem).wait()

    _prefetch(0, src_pa, dst_pa, typ_pa, sem_pa)

    def _zrow(k, carry):
        i = k // (IN_FEAT // LANES)
        j = lax.rem(k, IN_FEAT // LANES)
        rows_a[i, pl.ds(j * LANES, LANES)] = jnp.zeros((LANES,), jnp.float32)
        return carry

    lax.fori_loop(0, CHUNK * (IN_FEAT // LANES), _zrow, None)
    for k in range(ROWS_PER_SUB // CHUNK):
        pltpu.async_copy(
            rows_a, hacc.at[pl.ds(sid * ROWS_PER_SUB + k * CHUNK, CHUNK)], sem_sa)
    for k in range(ROWS_PER_SUB // CHUNK):
        pltpu.make_async_copy(
            rows_a, hacc.at[pl.ds(sid * ROWS_PER_SUB + k * CHUNK, CHUNK)], sem_sa).wait()
    plsc.subcore_barrier()

    def _prep_g(gidx_v, dst_v, c, dst_small, rows, sem):
        for i in range(CHUNK // LANES):
            dst_small[pl.ds(i * LANES, LANES)] = dst_v[pl.ds(c * CHUNK + i * LANES, LANES)]
        pltpu.async_copy(
            xw_hbm.at[gidx_v.at[pl.ds(c * CHUNK, CHUNK)]], rows, sem)

    def _wait_g(gidx_v, rows, sem):
        pltpu.make_async_copy(xw_hbm.at[gidx_v.at[pl.ds(0, CHUNK)]], rows, sem).wait()

    def _start_s(dst_small, rows, sem):
        pltpu.async_copy(rows, hacc.at[dst_small], sem, add=True)

    def _wait_s(dst_small, rows, sem):
        pltpu.make_async_copy(rows, hacc.at[dst_small], sem).wait()

    def _gidx(src_v, typ_v):
        def body(i, c2):
            sl = pl.ds(i * LANES, LANES)
            src_v[sl] = typ_v[sl] * N + src_v[sl]
            return c2
        lax.fori_loop(0, SBLOCK // LANES, body, None)

    def _run_block(gidx_v, dst_v):
        bufs = [(dst_a, rows_a, sem_ga, sem_sa),
                (dst_b, rows_b, sem_gb, sem_sb),
                (dst_c, rows_c, sem_gc, sem_sc)]

        def g(c, i):
            ds, rw, sg, _ = bufs[i]
            _prep_g(gidx_v, dst_v, c, ds, rw, sg)

        def wg_s(c, i):
            ds, rw, sg, ss = bufs[i]
            _wait_g(gidx_v, rw, sg)
            _start_s(ds, rw, ss)

        def ws(i):
            ds, rw, _, ss = bufs[i]
            _wait_s(ds, rw, ss)

        g(0, 0)
        g(1, 1)
        wg_s(0, 0)
        g(2, 2)
        wg_s(1, 1)
        ws(0)
        g(3, 0)

        def _pipe(k, c2):
            c = 2 + 3 * k
            wg_s(c, 2)
            ws(1)
            g(c + 2, 1)
            wg_s(c + 1, 0)
            ws(2)
            g(c + 3, 2)
            wg_s(c + 2, 1)
            ws(0)
            g(c + 4, 0)
            return c2

        lax.fori_loop(0, (CHUNKS_PER_SBLOCK - 4) // 3, _pipe, None)

        wg_s(CHUNKS_PER_SBLOCK - 2, 2)
        wg_s(CHUNKS_PER_SBLOCK - 1, 0)
        ws(1)
        ws(2)
        ws(0)

    def _block_pair(p, carry):
        b0 = 2 * p
        _pwait(src_pa, dst_pa, typ_pa, sem_pa)
        _prefetch(b0 + 1, src_pb, dst_pb, typ_pb, sem_pb)
        _gidx(src_pa, typ_pa)
        _run_block(src_pa, dst_pa)

        _pwait(src_pb, dst_pb, typ_pb, sem_pb)
        _prefetch(b0 + 2, src_pa, dst_pa, typ_pa, sem_pa)
        _gidx(src_pb, typ_pb)
        _run_block(src_pb, dst_pb)
        return carry

    lax.fori_loop(0, (NBLOCKS - 1) // 2, _block_pair, None)
    _pwait(src_pa, dst_pa, typ_pa, sem_pa)
    _gidx(src_pa, typ_pa)
    _run_block(src_pa, dst_pa)
    plsc.subcore_barrier()

    pltpu.sync_copy(hacc.at[pl.ds(sid * ROWS_PER_SUB, ROWS_PER_SUB)],
                    out_hbm.at[cid, pl.ds(sid * ROWS_PER_SUB, ROWS_PER_SUB)])


def _edge_sc(xw_flat, ei_flat, edge_type):
    call = pl.kernel(
        _edge_body,
        out_type=jax.ShapeDtypeStruct((NC, ACC_ROWS, OUT_FEAT), jnp.float32),
        mesh=plsc.VectorSubcoreMesh(
            core_axis_name="c", subcore_axis_name="s",
            num_cores=NC, num_subcores=NS),
        scratch_types=[
            pltpu.VMEM((SBLOCK,), jnp.int32),
            pltpu.VMEM((SBLOCK,), jnp.int32),
            pltpu.VMEM((SBLOCK,), jnp.int32),
            pltpu.VMEM((SBLOCK,), jnp.int32),
            pltpu.VMEM((SBLOCK,), jnp.int32),
            pltpu.VMEM((SBLOCK,), jnp.int32),
            pltpu.VMEM((CHUNK, OUT_FEAT), jnp.float32),
            pltpu.VMEM((CHUNK, OUT_FEAT), jnp.float32),
            pltpu.VMEM((CHUNK, OUT_FEAT), jnp.float32),
            pltpu.VMEM((CHUNK,), jnp.int32),
            pltpu.VMEM((CHUNK,), jnp.int32),
            pltpu.VMEM((CHUNK,), jnp.int32),
            pltpu.VMEM_SHARED((ACC_ROWS, OUT_FEAT), jnp.float32),
            pltpu.SemaphoreType.DMA,
            pltpu.SemaphoreType.DMA,
            pltpu.SemaphoreType.DMA,
            pltpu.SemaphoreType.DMA,
            pltpu.SemaphoreType.DMA,
            pltpu.SemaphoreType.DMA,
            pltpu.SemaphoreType.DMA,
            pltpu.SemaphoreType.DMA,
        ],
    )
    return call(xw_flat, ei_flat, edge_type)


_CB = 2000


def _combine_body(p_ref, o_ref):
    o_ref[...] = p_ref[0] + p_ref[1]


def _combine(part):
    return pl.pallas_call(
        _combine_body,
        grid=(N // _CB,),
        in_specs=[pl.BlockSpec((NC, _CB, OUT_FEAT), lambda i: (0, i, 0))],
        out_specs=pl.BlockSpec((_CB, OUT_FEAT), lambda i: (i, 0)),
        out_shape=jax.ShapeDtypeStruct((N, OUT_FEAT), jnp.float32),
    )(part)


def kernel(x, edge_index, edge_type, weight, w_comp):
    w2d = weight.reshape(NUM_BASES * IN_FEAT, OUT_FEAT)
    xw = _xw_table(x, w2d, w_comp)
    xw_flat = xw.reshape(NUM_RELS * N, OUT_FEAT)
    ei_flat = edge_index.reshape(2 * E)
    part = _edge_sc(xw_flat, ei_flat, edge_type)
    return _combine(part)


# speedup vs baseline: 1.4268x; 1.0085x over previous
import functools

import jax
import jax.numpy as jnp
from jax import lax
from jax.experimental import pallas as pl
from jax.experimental.pallas import tpu as pltpu
from jax.experimental.pallas import tpu_sc as plsc

N = 10000
E = 320000
IN_FEAT = 128
OUT_FEAT = 128
NUM_RELS = 8
NUM_BASES = 4

NC = 2
NS = 16
NW = NC * NS
LANES = 16

EDGES_PER_WORKER = E // NW
CHUNK = 80
SBLOCK = 2000
CHUNKS_PER_SBLOCK = SBLOCK // CHUNK
NBLOCKS = EDGES_PER_WORKER // SBLOCK
ACC_ROWS = 10240
ROWS_PER_SUB = ACC_ROWS // NS


_BN = 2000


def _xw_body(x_ref, w2d_ref, wc_ref, out_ref, wbig_ref):
    @pl.when(pl.program_id(0) == 0)
    def _build_w():
        KK, JJ = NUM_RELS * IN_FEAT, NUM_BASES * IN_FEAT
        ki = lax.broadcasted_iota(jnp.int32, (KK, JJ), 0)
        ji = lax.broadcasted_iota(jnp.int32, (KK, JJ), 1)
        eq = ((ki // NUM_RELS) == (ji // NUM_BASES)).astype(jnp.float32)
        ar = lax.broadcasted_iota(jnp.int32, (KK, NUM_RELS), 0)
        ac = lax.broadcasted_iota(jnp.int32, (KK, NUM_RELS), 1)
        amat = (lax.rem(ar, NUM_RELS) == ac).astype(jnp.float32)
        wr = lax.broadcasted_iota(jnp.int32, (NUM_RELS, JJ), 0)
        wj = lax.broadcasted_iota(jnp.int32, (NUM_RELS, JJ), 1)
        wjb = lax.rem(wj, NUM_BASES)
        wcb = jnp.zeros((NUM_RELS, JJ), jnp.float32)
        for r in range(NUM_RELS):
            for b in range(NUM_BASES):
                wcb = wcb + jnp.where((wr == r) & (wjb == b), wc_ref[r, b], 0.0)
        tiled = jnp.dot(amat, wcb, preferred_element_type=jnp.float32)
        mmat = eq * tiled
        wbig_ref[...] = jnp.dot(
            mmat, w2d_ref[...], preferred_element_type=jnp.float32
        ).astype(jnp.bfloat16)

    x = x_ref[...].astype(jnp.bfloat16)
    for r in range(NUM_RELS):
        out_ref[r] = jnp.dot(
            x, wbig_ref[IN_FEAT * r:IN_FEAT * (r + 1), :],
            preferred_element_type=jnp.float32)


def _xw_table(x, w2d, w_comp):
    return pl.pallas_call(
        _xw_body,
        grid=(N // _BN,),
        in_specs=[
            pl.BlockSpec((_BN, IN_FEAT), lambda i: (i, 0)),
            pl.BlockSpec((NUM_BASES * IN_FEAT, OUT_FEAT), lambda i: (0, 0)),
            pl.BlockSpec(memory_space=pltpu.SMEM),
        ],
        out_specs=pl.BlockSpec((NUM_RELS, _BN, OUT_FEAT), lambda i: (0, i, 0)),
        out_shape=jax.ShapeDtypeStruct((NUM_RELS, N, OUT_FEAT), jnp.float32),
        scratch_shapes=[pltpu.VMEM((NUM_RELS * IN_FEAT, OUT_FEAT), jnp.bfloat16)],
    )(x, w2d, w_comp)


def _edge_body(xw_hbm, ei_hbm, et_hbm, out_hbm,
               src_pa, src_pb, dst_pa, dst_pb, typ_pa, typ_pb,
               rows_a, rows_b, rows_c, dst_a, dst_b, dst_c, hacc,
               sem_pa, sem_pb, sem_ga, sem_gb, sem_gc, sem_sa, sem_sb, sem_sc):
    cid = lax.axis_index("c")
    sid = lax.axis_index("s")
    wid = cid * NS + sid
    base = wid * EDGES_PER_WORKER

    def _prefetch(b, src_v, dst_v, typ_v, sem):
        off = base + b * SBLOCK
        pltpu.async_copy(ei_hbm.at[pl.ds(off, SBLOCK)], src_v, sem)
        pltpu.async_copy(ei_hbm.at[pl.ds(E + off, SBLOCK)], dst_v, sem)
        pltpu.async_copy(et_hbm.at[pl.ds(off, SBLOCK)], typ_v, sem)

    def _pwait(src_v, dst_v, typ_v, sem):
        pltpu.make_async_copy(ei_hbm.at[pl.ds(0, SBLOCK)], src_v, sem).wait()
        pltpu.make_async_copy(ei_hbm.at[pl.ds(0, SBLOCK)], dst_v, sem).wait()
        pltpu.make_async_copy(et_hbm.at[pl.ds(0, SBLOCK)], typ_v, sem).wait()

    _prefetch(0, src_pa, dst_pa, typ_pa, sem_pa)

    def _zrow(k, carry):
        i = k // (IN_FEAT // LANES)
        j = lax.rem(k, IN_FEAT // LANES)
        rows_a[i, pl.ds(j * LANES, LANES)] = jnp.zeros((LANES,), jnp.float32)
        return carry

    lax.fori_loop(0, CHUNK * (IN_FEAT // LANES), _zrow, None)
    for k in range(ROWS_PER_SUB // CHUNK):
        pltpu.async_copy(
            rows_a, hacc.at[pl.ds(sid * ROWS_PER_SUB + k * CHUNK, CHUNK)], sem_sa)
    for k in range(ROWS_PER_SUB // CHUNK):
        pltpu.make_async_copy(
            rows_a, hacc.at[pl.ds(sid * ROWS_PER_SUB + k * CHUNK, CHUNK)], sem_sa).wait()
    plsc.subcore_barrier()

    def _prep_g(gidx_v, dst_v, c, dst_small, rows, sem):
        for i in range(CHUNK // LANES):
            dst_small[pl.ds(i * LANES, LANES)] = dst_v[pl.ds(c * CHUNK + i * LANES, LANES)]
        pltpu.async_copy(
            xw_hbm.at[gidx_v.at[pl.ds(c * CHUNK, CHUNK)]], rows, sem)

    def _wait_g(gidx_v, rows, sem):
        pltpu.make_async_copy(xw_hbm.at[gidx_v.at[pl.ds(0, CHUNK)]], rows, sem).wait()

    def _start_s(dst_small, rows, sem):
        pltpu.async_copy(rows, hacc.at[dst_small], sem, add=True)

    def _wait_s(dst_small, rows, sem):
        pltpu.make_async_copy(rows, hacc.at[dst_small], sem).wait()

    def _gidx(src_v, typ_v):
        def body(i, c2):
            sl = pl.ds(i * LANES, LANES)
            src_v[sl] = typ_v[sl] * N + src_v[sl]
            return c2
        lax.fori_loop(0, SBLOCK // LANES, body, None)

    bufs = [(dst_a, rows_a, sem_ga, sem_sa),
            (dst_b, rows_b, sem_gb, sem_sb),
            (dst_c, rows_c, sem_gc, sem_sc)]

    def _ws(i):
        ds, rw, _, ss = bufs[i]
        _wait_s(ds, rw, ss)

    def _run_block(gidx_v, dst_v, first):
        def g(c, i):
            ds, rw, sg, _ = bufs[i]
            _prep_g(gidx_v, dst_v, c, ds, rw, sg)

        def wg_s(c, i):
            ds, rw, sg, ss = bufs[i]
            _wait_g(gidx_v, rw, sg)
            _start_s(ds, rw, ss)

        ws = _ws

        if not first:
            ws(0)
        g(0, 0)
        if not first:
            ws(1)
        g(1, 1)
        wg_s(0, 0)
        if not first:
            ws(2)
        g(2, 2)
        wg_s(1, 1)
        ws(0)
        g(3, 0)

        def _pipe(k, c2):
            c = 2 + 3 * k
            wg_s(c, 2)
            ws(1)
            g(c + 2, 1)
            wg_s(c + 1, 0)
            ws(2)
            g(c + 3, 2)
            wg_s(c + 2, 1)
            ws(0)
            g(c + 4, 0)
            return c2

        lax.fori_loop(0, (CHUNKS_PER_SBLOCK - 4) // 3, _pipe, None)

        wg_s(CHUNKS_PER_SBLOCK - 2, 2)
        wg_s(CHUNKS_PER_SBLOCK - 1, 0)

    _pwait(src_pa, dst_pa, typ_pa, sem_pa)
    _prefetch(1, src_pb, dst_pb, typ_pb, sem_pb)
    _gidx(src_pa, typ_pa)
    _run_block(src_pa, dst_pa, True)

    def _block_pair(p, carry):
        b1 = 2 * p + 1
        _pwait(src_pb, dst_pb, typ_pb, sem_pb)
        _prefetch(b1 + 1, src_pa, dst_pa, typ_pa, sem_pa)
        _gidx(src_pb, typ_pb)
        _run_block(src_pb, dst_pb, False)

        _pwait(src_pa, dst_pa, typ_pa, sem_pa)

        @pl.when(b1 + 2 < NBLOCKS)
        def _():
            _prefetch(b1 + 2, src_pb, dst_pb, typ_pb, sem_pb)

        _gidx(src_pa, typ_pa)
        _run_block(src_pa, dst_pa, False)
        return carry

    lax.fori_loop(0, (NBLOCKS - 1) // 2, _block_pair, None)
    _ws(1)
    _ws(2)
    _ws(0)
    plsc.subcore_barrier()

    pltpu.sync_copy(hacc.at[pl.ds(sid * ROWS_PER_SUB, ROWS_PER_SUB)],
                    out_hbm.at[cid, pl.ds(sid * ROWS_PER_SUB, ROWS_PER_SUB)])


def _edge_sc(xw_flat, ei_flat, edge_type):
    call = pl.kernel(
        _edge_body,
        out_type=jax.ShapeDtypeStruct((NC, ACC_ROWS, OUT_FEAT), jnp.float32),
        mesh=plsc.VectorSubcoreMesh(
            core_axis_name="c", subcore_axis_name="s",
            num_cores=NC, num_subcores=NS),
        scratch_types=[
            pltpu.VMEM((SBLOCK,), jnp.int32),
            pltpu.VMEM((SBLOCK,), jnp.int32),
            pltpu.VMEM((SBLOCK,), jnp.int32),
            pltpu.VMEM((SBLOCK,), jnp.int32),
            pltpu.VMEM((SBLOCK,), jnp.int32),
            pltpu.VMEM((SBLOCK,), jnp.int32),
            pltpu.VMEM((CHUNK, OUT_FEAT), jnp.float32),
            pltpu.VMEM((CHUNK, OUT_FEAT), jnp.float32),
            pltpu.VMEM((CHUNK, OUT_FEAT), jnp.float32),
            pltpu.VMEM((CHUNK,), jnp.int32),
            pltpu.VMEM((CHUNK,), jnp.int32),
            pltpu.VMEM((CHUNK,), jnp.int32),
            pltpu.VMEM_SHARED((ACC_ROWS, OUT_FEAT), jnp.float32),
            pltpu.SemaphoreType.DMA,
            pltpu.SemaphoreType.DMA,
            pltpu.SemaphoreType.DMA,
            pltpu.SemaphoreType.DMA,
            pltpu.SemaphoreType.DMA,
            pltpu.SemaphoreType.DMA,
            pltpu.SemaphoreType.DMA,
            pltpu.SemaphoreType.DMA,
        ],
    )
    return call(xw_flat, ei_flat, edge_type)


_CB = 2000


def _combine_body(p_ref, o_ref):
    o_ref[...] = p_ref[0] + p_ref[1]


def _combine(part):
    return pl.pallas_call(
        _combine_body,
        grid=(N // _CB,),
        in_specs=[pl.BlockSpec((NC, _CB, OUT_FEAT), lambda i: (0, i, 0))],
        out_specs=pl.BlockSpec((_CB, OUT_FEAT), lambda i: (i, 0)),
        out_shape=jax.ShapeDtypeStruct((N, OUT_FEAT), jnp.float32),
    )(part)


def kernel(x, edge_index, edge_type, weight, w_comp):
    w2d = weight.reshape(NUM_BASES * IN_FEAT, OUT_FEAT)
    xw = _xw_table(x, w2d, w_comp)
    xw_flat = xw.reshape(NUM_RELS * N, OUT_FEAT)
    ei_flat = edge_index.reshape(2 * E)
    part = _edge_sc(xw_flat, ei_flat, edge_type)
    return _combine(part)


# speedup vs baseline: 1.4276x; 1.0006x over previous
import jax
import jax.numpy as jnp
from jax import lax
from jax.experimental import pallas as pl
from jax.experimental.pallas import tpu as pltpu
from jax.experimental.pallas import tpu_sc as plsc

N = 10000
E = 320000
IN_FEAT = 128
OUT_FEAT = 128
NUM_RELS = 8
NUM_BASES = 4

NC = 2
NS = 16
NW = NC * NS
LANES = 16

EDGES_PER_WORKER = E // NW
CHUNK = 80
SBLOCK = 2000
CHUNKS_PER_SBLOCK = SBLOCK // CHUNK
NBLOCKS = EDGES_PER_WORKER // SBLOCK
ACC_ROWS = 10240
ROWS_PER_SUB = ACC_ROWS // NS


_BN = 2000


def _xw_body(x_ref, w2d_ref, wc_ref, out_ref, wbig_ref):
    @pl.when(pl.program_id(0) == 0)
    def _build_w():
        KK, JJ = NUM_RELS * IN_FEAT, NUM_BASES * IN_FEAT
        ki = lax.broadcasted_iota(jnp.int32, (KK, JJ), 0)
        ji = lax.broadcasted_iota(jnp.int32, (KK, JJ), 1)
        eq = ((ki // NUM_RELS) == (ji // NUM_BASES)).astype(jnp.float32)
        ar = lax.broadcasted_iota(jnp.int32, (KK, NUM_RELS), 0)
        ac = lax.broadcasted_iota(jnp.int32, (KK, NUM_RELS), 1)
        amat = (lax.rem(ar, NUM_RELS) == ac).astype(jnp.float32)
        wr = lax.broadcasted_iota(jnp.int32, (NUM_RELS, JJ), 0)
        wj = lax.broadcasted_iota(jnp.int32, (NUM_RELS, JJ), 1)
        wjb = lax.rem(wj, NUM_BASES)
        wcb = jnp.zeros((NUM_RELS, JJ), jnp.float32)
        for r in range(NUM_RELS):
            for b in range(NUM_BASES):
                wcb = wcb + jnp.where((wr == r) & (wjb == b), wc_ref[r, b], 0.0)
        tiled = jnp.dot(amat, wcb, preferred_element_type=jnp.float32)
        mmat = eq * tiled
        wbig_ref[...] = jnp.dot(
            mmat, w2d_ref[...], preferred_element_type=jnp.float32
        ).astype(jnp.bfloat16)

    x = x_ref[...].astype(jnp.bfloat16)
    for r in range(NUM_RELS):
        out_ref[r] = jnp.dot(
            x, wbig_ref[IN_FEAT * r:IN_FEAT * (r + 1), :],
            preferred_element_type=jnp.float32)


def _xw_table(x, w2d, w_comp):
    return pl.pallas_call(
        _xw_body,
        grid=(N // _BN,),
        in_specs=[
            pl.BlockSpec((_BN, IN_FEAT), lambda i: (i, 0)),
            pl.BlockSpec((NUM_BASES * IN_FEAT, OUT_FEAT), lambda i: (0, 0)),
            pl.BlockSpec(memory_space=pltpu.SMEM),
        ],
        out_specs=pl.BlockSpec((NUM_RELS, _BN, OUT_FEAT), lambda i: (0, i, 0)),
        out_shape=jax.ShapeDtypeStruct((NUM_RELS, N, OUT_FEAT), jnp.float32),
        scratch_shapes=[pltpu.VMEM((NUM_RELS * IN_FEAT, OUT_FEAT), jnp.bfloat16)],
    )(x, w2d, w_comp)


def _edge_body(xw_hbm, ei_hbm, et_hbm, out_hbm,
               src_pa, src_pb, dst_pa, dst_pb, typ_pa, typ_pb,
               rows_a, rows_b, rows_c, dst_a, dst_b, dst_c, hacc,
               sem_pa, sem_pb, sem_ga, sem_gb, sem_gc, sem_sa, sem_sb, sem_sc):
    cid = lax.axis_index("c")
    sid = lax.axis_index("s")
    wid = cid * NS + sid
    base = wid * EDGES_PER_WORKER

    def _prefetch(b, src_v, dst_v, typ_v, sem):
        off = base + b * SBLOCK
        pltpu.async_copy(ei_hbm.at[pl.ds(off, SBLOCK)], src_v, sem)
        pltpu.async_copy(ei_hbm.at[pl.ds(E + off, SBLOCK)], dst_v, sem)
        pltpu.async_copy(et_hbm.at[pl.ds(off, SBLOCK)], typ_v, sem)

    def _pwait(src_v, dst_v, typ_v, sem):
        pltpu.make_async_copy(ei_hbm.at[pl.ds(0, SBLOCK)], src_v, sem).wait()
        pltpu.make_async_copy(ei_hbm.at[pl.ds(0, SBLOCK)], dst_v, sem).wait()
        pltpu.make_async_copy(et_hbm.at[pl.ds(0, SBLOCK)], typ_v, sem).wait()

    _prefetch(0, src_pa, dst_pa, typ_pa, sem_pa)

    def _zrow(k, carry):
        i = k // (IN_FEAT // LANES)
        j = lax.rem(k, IN_FEAT // LANES)
        rows_a[i, pl.ds(j * LANES, LANES)] = jnp.zeros((LANES,), jnp.float32)
        return carry

    lax.fori_loop(0, CHUNK * (IN_FEAT // LANES), _zrow, None)
    for k in range(ROWS_PER_SUB // CHUNK):
        pltpu.async_copy(
            rows_a, hacc.at[pl.ds(sid * ROWS_PER_SUB + k * CHUNK, CHUNK)], sem_sa)
    for k in range(ROWS_PER_SUB // CHUNK):
        pltpu.make_async_copy(
            rows_a, hacc.at[pl.ds(sid * ROWS_PER_SUB + k * CHUNK, CHUNK)], sem_sa).wait()
    plsc.subcore_barrier()

    def _prep_g(gidx_v, dst_v, c, dst_small, rows, sem):
        for i in range(CHUNK // LANES):
            dst_small[pl.ds(i * LANES, LANES)] = dst_v[pl.ds(c * CHUNK + i * LANES, LANES)]
        pltpu.async_copy(
            xw_hbm.at[gidx_v.at[pl.ds(c * CHUNK, CHUNK)]], rows, sem)

    def _wait_g(gidx_v, rows, sem):
        pltpu.make_async_copy(xw_hbm.at[gidx_v.at[pl.ds(0, CHUNK)]], rows, sem).wait()

    def _start_s(dst_small, rows, sem):
        pltpu.async_copy(rows, hacc.at[dst_small], sem, add=True)

    def _wait_s(dst_small, rows, sem):
        pltpu.make_async_copy(rows, hacc.at[dst_small], sem).wait()

    def _gidx(src_v, typ_v):
        def body(i, c2):
            sl = pl.ds(i * LANES, LANES)
            src_v[sl] = typ_v[sl] * N + src_v[sl]
            return c2
        lax.fori_loop(0, SBLOCK // LANES, body, None)

    bufs = [(dst_a, rows_a, sem_ga, sem_sa),
            (dst_b, rows_b, sem_gb, sem_sb),
            (dst_c, rows_c, sem_gc, sem_sc)]

    def _ws(i):
        ds, rw, _, ss = bufs[i]
        _wait_s(ds, rw, ss)

    def _run_block(gidx_v, dst_v, first):
        def g(c, i):
            ds, rw, sg, _ = bufs[i]
            _prep_g(gidx_v, dst_v, c, ds, rw, sg)

        def wg_s(c, i):
            ds, rw, sg, ss = bufs[i]
            _wait_g(gidx_v, rw, sg)
            _start_s(ds, rw, ss)

        ws = _ws

        if not first:
            ws(0)
        g(0, 0)
        if not first:
            ws(1)
        g(1, 1)
        wg_s(0, 0)
        if not first:
            ws(2)
        g(2, 2)
        wg_s(1, 1)
        ws(0)
        g(3, 0)

        def _pipe(k, c2):
            c = 2 + 3 * k
            wg_s(c, 2)
            ws(1)
            g(c + 2, 1)
            wg_s(c + 1, 0)
            ws(2)
            g(c + 3, 2)
            wg_s(c + 2, 1)
            ws(0)
            g(c + 4, 0)
            return c2

        lax.fori_loop(0, (CHUNKS_PER_SBLOCK - 4) // 3, _pipe, None)

        wg_s(CHUNKS_PER_SBLOCK - 2, 2)
        wg_s(CHUNKS_PER_SBLOCK - 1, 0)

    _pwait(src_pa, dst_pa, typ_pa, sem_pa)
    _prefetch(1, src_pb, dst_pb, typ_pb, sem_pb)
    _gidx(src_pa, typ_pa)
    _run_block(src_pa, dst_pa, True)

    def _block_pair(p, carry):
        b1 = 2 * p + 1
        _pwait(src_pb, dst_pb, typ_pb, sem_pb)
        _prefetch(b1 + 1, src_pa, dst_pa, typ_pa, sem_pa)
        _gidx(src_pb, typ_pb)
        _run_block(src_pb, dst_pb, False)

        _pwait(src_pa, dst_pa, typ_pa, sem_pa)

        @pl.when(b1 + 2 < NBLOCKS)
        def _():
            _prefetch(b1 + 2, src_pb, dst_pb, typ_pb, sem_pb)

        _gidx(src_pa, typ_pa)
        _run_block(src_pa, dst_pa, False)
        return carry

    lax.fori_loop(0, (NBLOCKS - 1) // 2, _block_pair, None)
    _ws(1)
    _ws(2)
    _ws(0)
    plsc.subcore_barrier()

    pltpu.sync_copy(hacc.at[pl.ds(sid * ROWS_PER_SUB, ROWS_PER_SUB)],
                    out_hbm.at[cid, pl.ds(sid * ROWS_PER_SUB, ROWS_PER_SUB)])


def _edge_sc(xw_flat, ei_flat, edge_type):
    call = pl.kernel(
        _edge_body,
        out_type=jax.ShapeDtypeStruct((NC, ACC_ROWS, OUT_FEAT), jnp.float32),
        mesh=plsc.VectorSubcoreMesh(
            core_axis_name="c", subcore_axis_name="s",
            num_cores=NC, num_subcores=NS),
        scratch_types=[
            pltpu.VMEM((SBLOCK,), jnp.int32),
            pltpu.VMEM((SBLOCK,), jnp.int32),
            pltpu.VMEM((SBLOCK,), jnp.int32),
            pltpu.VMEM((SBLOCK,), jnp.int32),
            pltpu.VMEM((SBLOCK,), jnp.int32),
            pltpu.VMEM((SBLOCK,), jnp.int32),
            pltpu.VMEM((CHUNK, OUT_FEAT), jnp.float32),
            pltpu.VMEM((CHUNK, OUT_FEAT), jnp.float32),
            pltpu.VMEM((CHUNK, OUT_FEAT), jnp.float32),
            pltpu.VMEM((CHUNK,), jnp.int32),
            pltpu.VMEM((CHUNK,), jnp.int32),
            pltpu.VMEM((CHUNK,), jnp.int32),
            pltpu.VMEM_SHARED((ACC_ROWS, OUT_FEAT), jnp.float32),
            pltpu.SemaphoreType.DMA,
            pltpu.SemaphoreType.DMA,
            pltpu.SemaphoreType.DMA,
            pltpu.SemaphoreType.DMA,
            pltpu.SemaphoreType.DMA,
            pltpu.SemaphoreType.DMA,
            pltpu.SemaphoreType.DMA,
            pltpu.SemaphoreType.DMA,
        ],
    )
    return call(xw_flat, ei_flat, edge_type)


_CB = 2000


def _combine_body(p_ref, o_ref):
    o_ref[...] = p_ref[0] + p_ref[1]


def _combine(part):
    return pl.pallas_call(
        _combine_body,
        grid=(N // _CB,),
        in_specs=[pl.BlockSpec((NC, _CB, OUT_FEAT), lambda i: (0, i, 0))],
        out_specs=pl.BlockSpec((_CB, OUT_FEAT), lambda i: (i, 0)),
        out_shape=jax.ShapeDtypeStruct((N, OUT_FEAT), jnp.float32),
    )(part)


def kernel(x, edge_index, edge_type, weight, w_comp):
    w2d = weight.reshape(NUM_BASES * IN_FEAT, OUT_FEAT)
    xw = _xw_table(x, w2d, w_comp)
    xw_flat = xw.reshape(NUM_RELS * N, OUT_FEAT)
    ei_flat = edge_index.reshape(2 * E)
    part = _edge_sc(xw_flat, ei_flat, edge_type)
    return _combine(part)
